# Initial kernel scaffold; baseline (speedup 1.0000x reference)
import jax
import jax.numpy as jnp
from jax.experimental import pallas as pl


def kernel(x, edge_index, edge_attr, W1, b1, W2, b2, W3, b3, W4, b4, root, bias):
    raise NotImplementedError("write your pallas kernel here")


# speedup vs baseline: 3.7404x; 3.7404x over previous
import functools

import jax
import jax.numpy as jnp
from jax.experimental import pallas as pl
from jax.experimental.pallas import tpu as pltpu
from jax.experimental.pallas import tpu_sc as plsc

H = 20
OUT = 3

_SC_CORES = 2
_SC_SUBCORES = 16
_NW = _SC_CORES * _SC_SUBCORES
_CHUNK = 128


def _sc_mesh():
    return plsc.VectorSubcoreMesh(core_axis_name="c", subcore_axis_name="s")


def _gather_rows(table, idx):
    e = idx.shape[0]
    d = table.shape[1]
    n_chunks = e // _CHUNK
    n_it = -(-n_chunks // _NW)

    @functools.partial(
        pl.kernel,
        out_type=jax.ShapeDtypeStruct((e, d), jnp.float32),
        mesh=_sc_mesh(),
        scratch_types=[
            pltpu.VMEM((_CHUNK,), jnp.int32),
            pltpu.VMEM((_CHUNK, d), jnp.float32),
            pltpu.SemaphoreType.DMA,
        ],
    )
    def k(table_hbm, idx_hbm, out_hbm, idx_v, rows_v, sem):
        cid = jax.lax.axis_index("c")
        sid = jax.lax.axis_index("s")
        wid = cid * _SC_SUBCORES + sid

        @pl.loop(0, n_it)
        def _(j):
            c = wid + j * _NW

            @pl.when(c < n_chunks)
            def _():
                off = c * _CHUNK
                pltpu.sync_copy(idx_hbm.at[pl.ds(off, _CHUNK)], idx_v)
                pltpu.async_copy(table_hbm.at[idx_v], rows_v, sem).wait()
                pltpu.sync_copy(rows_v, out_hbm.at[pl.ds(off, _CHUNK)])

    return k(table, idx)


def _edge_messages(ea, xj, umat, smat, w1, b1, w2, b2, w3, b3):
    e = ea.shape[0]
    be = 2560
    grid = (e // be,)

    def body(ea_ref, xj_ref, u_ref, s_ref, w1_ref, b1_ref, w2_ref, b2_ref,
             w3_ref, b3_ref, out_ref):
        a = ea_ref[...]
        h = jnp.maximum(
            jnp.dot(a, w1_ref[...], preferred_element_type=jnp.float32)
            + b1_ref[...], 0.0)
        h = jnp.maximum(
            jnp.dot(h, w2_ref[...], preferred_element_type=jnp.float32)
            + b2_ref[...], 0.0)
        h = jnp.maximum(
            jnp.dot(h, w3_ref[...], preferred_element_type=jnp.float32)
            + b3_ref[...], 0.0)
        g = jnp.dot(xj_ref[...], u_ref[...],
                    preferred_element_type=jnp.float32)
        hg = jnp.concatenate(
            [jnp.concatenate([h, h, h], axis=1) * g[:, 0:3 * H],
             g[:, 3 * H:3 * H + OUT],
             jnp.ones((be, 1), jnp.float32)], axis=1)
        out_ref[...] = jnp.dot(hg, s_ref[...],
                               preferred_element_type=jnp.float32)

    full = lambda arr: pl.BlockSpec(arr.shape, lambda i: (0,) * arr.ndim)
    return pl.pallas_call(
        body,
        grid=grid,
        in_specs=[
            pl.BlockSpec((be, ea.shape[1]), lambda i: (i, 0)),
            pl.BlockSpec((be, xj.shape[1]), lambda i: (i, 0)),
            full(umat), full(smat),
            full(w1), full(b1), full(w2), full(b2), full(w3), full(b3),
        ],
        out_specs=pl.BlockSpec((be, 4), lambda i: (i, 0)),
        out_shape=jax.ShapeDtypeStruct((e, 4), jnp.float32),
    )(ea, xj, umat, smat, w1, b1, w2, b2, w3, b3)


def _scatter_messages(msgt, dst, zeros_n):
    e = dst.shape[0]
    n = zeros_n.shape[0]
    n_chunks = e // _CHUNK
    n_it = -(-n_chunks // _NW)

    @functools.partial(
        pl.kernel,
        out_type=jax.ShapeDtypeStruct((_SC_CORES, 4, n), jnp.float32),
        mesh=_sc_mesh(),
        scratch_types=[
            pltpu.VMEM((_CHUNK,), jnp.int32),
            pltpu.VMEM((_CHUNK,), jnp.float32),
            pltpu.VMEM_SHARED((n,), jnp.float32),
            pltpu.VMEM_SHARED((n,), jnp.float32),
            pltpu.VMEM_SHARED((n,), jnp.float32),
            pltpu.VMEM_SHARED((n,), jnp.float32),
        ],
    )
    def k(msgt_hbm, dst_hbm, z_hbm, out_hbm, idx_v, val_v,
          acc0, acc1, acc2, acc3):
        cid = jax.lax.axis_index("c")
        sid = jax.lax.axis_index("s")
        wid = cid * _SC_SUBCORES + sid
        accs = [acc0, acc1, acc2, acc3]

        @pl.when(sid == 0)
        def _():
            for o in range(4):
                pltpu.sync_copy(z_hbm, accs[o])

        plsc.subcore_barrier()

        @pl.loop(0, n_it)
        def _(j):
            c = wid + j * _NW

            @pl.when(c < n_chunks)
            def _():
                off = c * _CHUNK
                pltpu.sync_copy(dst_hbm.at[pl.ds(off, _CHUNK)], idx_v)
                for o in range(4):
                    pltpu.sync_copy(msgt_hbm.at[o, pl.ds(off, _CHUNK)], val_v)
                    pltpu.sync_copy(val_v, accs[o].at[idx_v], add=True)

        plsc.subcore_barrier()

        @pl.when(sid == 0)
        def _():
            for o in range(4):
                pltpu.sync_copy(accs[o], out_hbm.at[cid, o])

    return k(msgt, dst, zeros_n)


def _finalize(parts, x, rootp, bias3):
    n = x.shape[0]

    def body(p_ref, x_ref, r_ref, b_ref, o_ref):
        s = p_ref[0] + p_ref[1]
        cnt = jnp.maximum(s[:, 3:4], 1.0)
        rt = jnp.dot(x_ref[...], r_ref[...],
                     preferred_element_type=jnp.float32)
        o_ref[...] = s[:, 0:OUT] / cnt + rt[:, 0:OUT] + b_ref[...]

    return pl.pallas_call(
        body,
        out_shape=jax.ShapeDtypeStruct((n, OUT), jnp.float32),
    )(parts, x, rootp, bias3)


def kernel(x, edge_index, edge_attr, W1, b1, W2, b2, W3, b3, W4, b4, root,
           bias):
    n, d = x.shape
    src = edge_index[0].astype(jnp.int32)
    dst = edge_index[1].astype(jnp.int32)

    u2 = W4.reshape(H, d, OUT).transpose(1, 2, 0).reshape(d, H * OUT)
    b4mat = b4.reshape(d, OUT)
    umat = jnp.concatenate(
        [u2, b4mat, jnp.zeros((d, 64 - 3 * H - OUT), jnp.float32)], axis=1)
    sm = jnp.zeros((64, 4), jnp.float32)
    for o in range(OUT):
        sm = sm.at[o * H:(o + 1) * H, o].set(1.0)
        sm = sm.at[3 * H + o, o].set(1.0)
    smat = sm.at[63, 3].set(1.0)
    rootp = jnp.pad(root, ((0, 0), (0, 1)))
    bias3 = bias.reshape(1, OUT)

    xj = _gather_rows(x, src)
    msg4 = _edge_messages(
        edge_attr, xj, umat, smat, W1, b1.reshape(1, H), W2, b2.reshape(1, H),
        W3, b3.reshape(1, H))
    msgt = msg4.T
    parts = _scatter_messages(msgt, dst, jnp.zeros((n,), jnp.float32))
    partsn = jnp.transpose(parts, (0, 2, 1))
    return _finalize(partsn, x, rootp, bias3)


# speedup vs baseline: 5.3498x; 1.4303x over previous
import functools

import jax
import jax.numpy as jnp
from jax.experimental import pallas as pl
from jax.experimental.pallas import tpu as pltpu
from jax.experimental.pallas import tpu_sc as plsc

H = 20
OUT = 3

_SC_CORES = 2
_SC_SUBCORES = 16
_NW = _SC_CORES * _SC_SUBCORES
_CHUNK = 128
_NBUF = 4


def _sc_mesh():
    return plsc.VectorSubcoreMesh(core_axis_name="c", subcore_axis_name="s")


def _gather_rows(table, idx3):
    per_w = idx3.shape[1]
    n_chunks = _NW * per_w
    d = table.shape[1]

    @functools.partial(
        pl.kernel,
        out_type=jax.ShapeDtypeStruct((n_chunks * _CHUNK, d), jnp.float32),
        mesh=_sc_mesh(),
        scratch_types=[pltpu.VMEM((per_w, _CHUNK), jnp.int32)]
        + [pltpu.VMEM((_CHUNK, d), jnp.float32) for _ in range(_NBUF)]
        + [pltpu.SemaphoreType.DMA for _ in range(2 * _NBUF)],
    )
    def k(table_hbm, idx_hbm, out_hbm, idx_v, *bufs_and_sems):
        bufs = bufs_and_sems[:_NBUF]
        gsem = bufs_and_sems[_NBUF:2 * _NBUF]
        wsem = bufs_and_sems[2 * _NBUF:]
        cid = jax.lax.axis_index("c")
        sid = jax.lax.axis_index("s")
        wid = cid * _SC_SUBCORES + sid
        base = wid * per_w
        pltpu.sync_copy(idx_hbm.at[wid], idx_v)

        n_rounds = -(-per_w // _NBUF)

        @pl.loop(0, n_rounds * _NBUF, step=_NBUF)
        def _(j):
            for b in range(_NBUF):
                c = j + b

                @pl.when(c < per_w)
                def _():
                    @pl.when(j > 0)
                    def _():
                        pltpu.make_async_copy(
                            bufs[b], out_hbm.at[pl.ds(0, _CHUNK)],
                            wsem[b]).wait()

                    pltpu.async_copy(
                        table_hbm.at[idx_v.at[c]], bufs[b], gsem[b])

            for b in range(_NBUF):
                c = j + b

                @pl.when(c < per_w)
                def _():
                    pltpu.make_async_copy(
                        table_hbm.at[pl.ds(0, _CHUNK)], bufs[b],
                        gsem[b]).wait()
                    pltpu.async_copy(
                        bufs[b], out_hbm.at[pl.ds((base + c) * _CHUNK,
                                                  _CHUNK)], wsem[b])

        last = (n_rounds - 1) * _NBUF
        for b in range(_NBUF):
            @pl.when(last + b < per_w)
            def _():
                pltpu.make_async_copy(
                    bufs[b], out_hbm.at[pl.ds(0, _CHUNK)], wsem[b]).wait()

    return k(table, idx3)


def _edge_messages(ea, xj, umat, smatt, w1, b1, w2, b2, w3, b3):
    e = ea.shape[0]
    be = 2560
    grid = (e // be,)

    def body(ea_ref, xj_ref, u_ref, s_ref, w1_ref, b1_ref, w2_ref, b2_ref,
             w3_ref, b3_ref, out_ref):
        a = ea_ref[...]
        h = jnp.maximum(
            jnp.dot(a, w1_ref[...], preferred_element_type=jnp.float32)
            + b1_ref[...], 0.0)
        h = jnp.maximum(
            jnp.dot(h, w2_ref[...], preferred_element_type=jnp.float32)
            + b2_ref[...], 0.0)
        h = jnp.maximum(
            jnp.dot(h, w3_ref[...], preferred_element_type=jnp.float32)
            + b3_ref[...], 0.0)
        g = jnp.dot(xj_ref[...], u_ref[...],
                    preferred_element_type=jnp.float32)
        hg = jnp.concatenate(
            [jnp.concatenate([h, h, h], axis=1) * g[:, 0:3 * H],
             g[:, 3 * H:3 * H + OUT],
             jnp.ones((be, 1), jnp.float32)], axis=1)
        out_ref[...] = jax.lax.dot_general(
            s_ref[...], hg, (((1,), (1,)), ((), ())),
            preferred_element_type=jnp.float32)

    full = lambda arr: pl.BlockSpec(arr.shape, lambda i: (0,) * arr.ndim)
    return pl.pallas_call(
        body,
        grid=grid,
        in_specs=[
            pl.BlockSpec((be, ea.shape[1]), lambda i: (i, 0)),
            pl.BlockSpec((be, xj.shape[1]), lambda i: (i, 0)),
            full(umat), full(smatt),
            full(w1), full(b1), full(w2), full(b2), full(w3), full(b3),
        ],
        out_specs=pl.BlockSpec((4, be), lambda i: (0, i)),
        out_shape=jax.ShapeDtypeStruct((4, e), jnp.float32),
    )(ea, xj, umat, smatt, w1, b1, w2, b2, w3, b3)


def _scatter_messages(msgt4, dst3, zeros_n):
    per_w = dst3.shape[1]
    n = zeros_n.shape[0]

    @functools.partial(
        pl.kernel,
        out_type=jax.ShapeDtypeStruct((_SC_CORES, 4, n), jnp.float32),
        mesh=_sc_mesh(),
        scratch_types=[
            pltpu.VMEM((per_w, _CHUNK), jnp.int32),
            pltpu.VMEM((4, per_w, _CHUNK), jnp.float32),
            pltpu.VMEM_SHARED((n,), jnp.float32),
            pltpu.VMEM_SHARED((n,), jnp.float32),
            pltpu.VMEM_SHARED((n,), jnp.float32),
            pltpu.VMEM_SHARED((n,), jnp.float32),
        ] + [pltpu.SemaphoreType.DMA for _ in range(4)],
    )
    def k(msgt_hbm, dst_hbm, z_hbm, out_hbm, idx_v, val_v,
          acc0, acc1, acc2, acc3, s0, s1, s2, s3):
        cid = jax.lax.axis_index("c")
        sid = jax.lax.axis_index("s")
        wid = cid * _SC_SUBCORES + sid
        base = wid * per_w
        accs = [acc0, acc1, acc2, acc3]
        sems = [s0, s1, s2, s3]

        @pl.when(sid == 0)
        def _():
            for o in range(4):
                pltpu.sync_copy(z_hbm, accs[o])

        pltpu.sync_copy(dst_hbm.at[wid], idx_v)
        for o in range(4):
            pltpu.sync_copy(msgt_hbm.at[o, wid], val_v.at[o])
        plsc.subcore_barrier()

        @pl.loop(0, per_w)
        def _(j):
            @pl.when(j > 0)
            def _():
                for o in range(4):
                    pltpu.make_async_copy(
                        val_v.at[o, 0], accs[o].at[idx_v.at[0]],
                        sems[o]).wait()
            for o in range(4):
                pltpu.async_copy(
                    val_v.at[o, j], accs[o].at[idx_v.at[j]], sems[o],
                    add=True)

        for o in range(4):
            pltpu.make_async_copy(
                val_v.at[o, 0], accs[o].at[idx_v.at[0]], sems[o]).wait()

        plsc.subcore_barrier()

        @pl.when(sid == 0)
        def _():
            for o in range(4):
                pltpu.sync_copy(accs[o], out_hbm.at[cid, o])

    return k(msgt4, dst3, zeros_n)


def _finalize(parts, x, rootp, bias3):
    n = x.shape[0]

    def body(p_ref, x_ref, r_ref, b_ref, o_ref):
        s = p_ref[0] + p_ref[1]
        cnt = jnp.maximum(s[:, 3:4], 1.0)
        rt = jnp.dot(x_ref[...], r_ref[...],
                     preferred_element_type=jnp.float32)
        o_ref[...] = s[:, 0:OUT] / cnt + rt[:, 0:OUT] + b_ref[...]

    return pl.pallas_call(
        body,
        out_shape=jax.ShapeDtypeStruct((n, OUT), jnp.float32),
    )(parts, x, rootp, bias3)


def kernel(x, edge_index, edge_attr, W1, b1, W2, b2, W3, b3, W4, b4, root,
           bias):
    n, d = x.shape
    e = edge_attr.shape[0]
    src = edge_index[0].astype(jnp.int32)
    dst = edge_index[1].astype(jnp.int32)

    n_chunks = -(-e // (_CHUNK * _NW)) * _NW
    per_w = n_chunks // _NW
    pad = n_chunks * _CHUNK - e
    spread = jnp.arange(pad, dtype=jnp.int32) % n
    src3 = jnp.concatenate([src, spread]).reshape(_NW, per_w, _CHUNK)
    dst3 = jnp.concatenate([dst, spread]).reshape(_NW, per_w, _CHUNK)

    u2 = W4.reshape(H, d, OUT).transpose(1, 2, 0).reshape(d, H * OUT)
    b4mat = b4.reshape(d, OUT)
    umat = jnp.concatenate(
        [u2, b4mat, jnp.zeros((d, 64 - 3 * H - OUT), jnp.float32)], axis=1)
    sm = jnp.zeros((64, 4), jnp.float32)
    for o in range(OUT):
        sm = sm.at[o * H:(o + 1) * H, o].set(1.0)
        sm = sm.at[3 * H + o, o].set(1.0)
    smatt = sm.at[63, 3].set(1.0).T
    rootp = jnp.pad(root, ((0, 0), (0, 1)))
    bias3 = bias.reshape(1, OUT)

    xj = _gather_rows(x, src3)
    msgt = _edge_messages(
        edge_attr, xj, umat, smatt, W1, b1.reshape(1, H), W2,
        b2.reshape(1, H), W3, b3.reshape(1, H))
    msgt4 = jnp.pad(msgt, ((0, 0), (0, pad))).reshape(
        4, _NW, per_w, _CHUNK)
    parts = _scatter_messages(msgt4, dst3, jnp.zeros((n,), jnp.float32))
    partsn = jnp.transpose(parts, (0, 2, 1))
    return _finalize(partsn, x, rootp, bias3)


# speedup vs baseline: 9.0535x; 1.6923x over previous
import functools

import jax
import jax.numpy as jnp
from jax.experimental import pallas as pl
from jax.experimental.pallas import tpu as pltpu
from jax.experimental.pallas import tpu_sc as plsc

H = 20
OUT = 3

_SC_CORES = 2
_SC_SUBCORES = 16
_NW = _SC_CORES * _SC_SUBCORES
_CHUNK = 128
_NBUF = 4


def _sc_mesh():
    return plsc.VectorSubcoreMesh(core_axis_name="c", subcore_axis_name="s")


def _gather_rows(table, idx3):
    per_w = idx3.shape[1]
    n_chunks = _NW * per_w
    d = table.shape[1]

    @functools.partial(
        pl.kernel,
        out_type=jax.ShapeDtypeStruct((n_chunks * _CHUNK, d), jnp.float32),
        mesh=_sc_mesh(),
        scratch_types=[pltpu.VMEM((per_w, _CHUNK), jnp.int32)]
        + [pltpu.VMEM((_CHUNK, d), jnp.float32) for _ in range(_NBUF)]
        + [pltpu.SemaphoreType.DMA for _ in range(2 * _NBUF)],
    )
    def k(table_hbm, idx_hbm, out_hbm, idx_v, *bufs_and_sems):
        bufs = bufs_and_sems[:_NBUF]
        gsem = bufs_and_sems[_NBUF:2 * _NBUF]
        wsem = bufs_and_sems[2 * _NBUF:]
        cid = jax.lax.axis_index("c")
        sid = jax.lax.axis_index("s")
        wid = cid * _SC_SUBCORES + sid
        base = wid * per_w
        pltpu.sync_copy(idx_hbm.at[wid], idx_v)

        n_rounds = -(-per_w // _NBUF)

        @pl.loop(0, n_rounds * _NBUF, step=_NBUF)
        def _(j):
            for b in range(_NBUF):
                c = j + b

                @pl.when(c < per_w)
                def _():
                    @pl.when(j > 0)
                    def _():
                        pltpu.make_async_copy(
                            bufs[b], out_hbm.at[pl.ds(0, _CHUNK)],
                            wsem[b]).wait()

                    pltpu.async_copy(
                        table_hbm.at[idx_v.at[c]], bufs[b], gsem[b])

            for b in range(_NBUF):
                c = j + b

                @pl.when(c < per_w)
                def _():
                    pltpu.make_async_copy(
                        table_hbm.at[pl.ds(0, _CHUNK)], bufs[b],
                        gsem[b]).wait()
                    pltpu.async_copy(
                        bufs[b], out_hbm.at[pl.ds((base + c) * _CHUNK,
                                                  _CHUNK)], wsem[b])

        last = (n_rounds - 1) * _NBUF
        for b in range(_NBUF):
            @pl.when(last + b < per_w)
            def _():
                pltpu.make_async_copy(
                    bufs[b], out_hbm.at[pl.ds(0, _CHUNK)], wsem[b]).wait()

    return k(table, idx3)


def _edge_messages(eat, xj, umatt, smatt, w1t, b1c, w2t, b2c, w3t, b3c):
    e = eat.shape[1]
    be = 2560
    grid = (e // be,)

    def body(ea_ref, xj_ref, u_ref, s_ref, w1_ref, b1_ref, w2_ref, b2_ref,
             w3_ref, b3_ref, out_ref):
        at = ea_ref[...]
        ht = jnp.maximum(
            jnp.dot(w1_ref[...], at, preferred_element_type=jnp.float32)
            + b1_ref[...], 0.0)
        ht = jnp.maximum(
            jnp.dot(w2_ref[...], ht, preferred_element_type=jnp.float32)
            + b2_ref[...], 0.0)
        ht = jnp.maximum(
            jnp.dot(w3_ref[...], ht, preferred_element_type=jnp.float32)
            + b3_ref[...], 0.0)
        gt = jax.lax.dot_general(
            u_ref[...], xj_ref[...], (((1,), (1,)), ((), ())),
            preferred_element_type=jnp.float32)
        hgt = jnp.concatenate(
            [jnp.concatenate([ht, ht, ht], axis=0) * gt[0:3 * H, :],
             gt[3 * H:3 * H + OUT, :],
             jnp.ones((1, be), jnp.float32)], axis=0)
        out_ref[...] = jnp.dot(s_ref[...], hgt,
                               preferred_element_type=jnp.float32)

    full = lambda arr: pl.BlockSpec(arr.shape, lambda i: (0,) * arr.ndim)
    return pl.pallas_call(
        body,
        grid=grid,
        in_specs=[
            pl.BlockSpec((eat.shape[0], be), lambda i: (0, i)),
            pl.BlockSpec((be, xj.shape[1]), lambda i: (i, 0)),
            full(umatt), full(smatt),
            full(w1t), full(b1c), full(w2t), full(b2c), full(w3t), full(b3c),
        ],
        out_specs=pl.BlockSpec((4, be), lambda i: (0, i)),
        out_shape=jax.ShapeDtypeStruct((4, e), jnp.float32),
    )(eat, xj, umatt, smatt, w1t, b1c, w2t, b2c, w3t, b3c)


def _scatter_messages(msgt4, dst3, zeros_n):
    per_w = dst3.shape[1]
    n = zeros_n.shape[0]

    @functools.partial(
        pl.kernel,
        out_type=jax.ShapeDtypeStruct((_SC_CORES, 4, n), jnp.float32),
        mesh=_sc_mesh(),
        scratch_types=[
            pltpu.VMEM((per_w, _CHUNK), jnp.int32),
            pltpu.VMEM((4, per_w, _CHUNK), jnp.float32),
            pltpu.VMEM_SHARED((n,), jnp.float32),
            pltpu.VMEM_SHARED((n,), jnp.float32),
            pltpu.VMEM_SHARED((n,), jnp.float32),
            pltpu.VMEM_SHARED((n,), jnp.float32),
        ] + [pltpu.SemaphoreType.DMA for _ in range(4)],
    )
    def k(msgt_hbm, dst_hbm, z_hbm, out_hbm, idx_v, val_v,
          acc0, acc1, acc2, acc3, s0, s1, s2, s3):
        cid = jax.lax.axis_index("c")
        sid = jax.lax.axis_index("s")
        wid = cid * _SC_SUBCORES + sid
        base = wid * per_w
        accs = [acc0, acc1, acc2, acc3]
        sems = [s0, s1, s2, s3]

        @pl.when(sid == 0)
        def _():
            for o in range(4):
                pltpu.sync_copy(z_hbm, accs[o])

        pltpu.sync_copy(dst_hbm.at[wid], idx_v)
        for o in range(4):
            pltpu.sync_copy(msgt_hbm.at[o, wid], val_v.at[o])
        plsc.subcore_barrier()

        @pl.loop(0, per_w)
        def _(j):
            @pl.when(j > 0)
            def _():
                for o in range(4):
                    pltpu.make_async_copy(
                        val_v.at[o, 0], accs[o].at[idx_v.at[0]],
                        sems[o]).wait()
            for o in range(4):
                pltpu.async_copy(
                    val_v.at[o, j], accs[o].at[idx_v.at[j]], sems[o],
                    add=True)

        for o in range(4):
            pltpu.make_async_copy(
                val_v.at[o, 0], accs[o].at[idx_v.at[0]], sems[o]).wait()

        plsc.subcore_barrier()

        @pl.when(sid == 0)
        def _():
            for o in range(4):
                pltpu.sync_copy(accs[o], out_hbm.at[cid, o])

    return k(msgt4, dst3, zeros_n)


def _finalize(parts, x, rootp, bias3):
    n = x.shape[0]

    def body(p_ref, x_ref, r_ref, b_ref, o_ref):
        s = p_ref[0] + p_ref[1]
        cnt = jnp.maximum(s[:, 3:4], 1.0)
        rt = jnp.dot(x_ref[...], r_ref[...],
                     preferred_element_type=jnp.float32)
        o_ref[...] = s[:, 0:OUT] / cnt + rt[:, 0:OUT] + b_ref[...]

    return pl.pallas_call(
        body,
        out_shape=jax.ShapeDtypeStruct((n, OUT), jnp.float32),
    )(parts, x, rootp, bias3)


def kernel(x, edge_index, edge_attr, W1, b1, W2, b2, W3, b3, W4, b4, root,
           bias):
    n, d = x.shape
    e = edge_attr.shape[0]
    src = edge_index[0].astype(jnp.int32)
    dst = edge_index[1].astype(jnp.int32)

    n_chunks = -(-e // (_CHUNK * _NW)) * _NW
    per_w = n_chunks // _NW
    pad = n_chunks * _CHUNK - e
    spread = jnp.arange(pad, dtype=jnp.int32) % n
    src3 = jnp.concatenate([src, spread]).reshape(_NW, per_w, _CHUNK)
    dst3 = jnp.concatenate([dst, spread]).reshape(_NW, per_w, _CHUNK)

    u2 = W4.reshape(H, d, OUT).transpose(1, 2, 0).reshape(d, H * OUT)
    b4mat = b4.reshape(d, OUT)
    umat = jnp.concatenate(
        [u2, b4mat, jnp.zeros((d, 64 - 3 * H - OUT), jnp.float32)], axis=1)
    sm = jnp.zeros((64, 4), jnp.float32)
    for o in range(OUT):
        sm = sm.at[o * H:(o + 1) * H, o].set(1.0)
        sm = sm.at[3 * H + o, o].set(1.0)
    smatt = sm.at[63, 3].set(1.0).T
    rootp = jnp.pad(root, ((0, 0), (0, 1)))
    bias3 = bias.reshape(1, OUT)

    xj = _gather_rows(x, src3)
    msgt = _edge_messages(
        edge_attr.T, xj, umat.T, smatt, W1.T, b1.reshape(H, 1), W2.T,
        b2.reshape(H, 1), W3.T, b3.reshape(H, 1))
    msgt4 = jnp.pad(msgt, ((0, 0), (0, pad))).reshape(
        4, _NW, per_w, _CHUNK)
    parts = _scatter_messages(msgt4, dst3, jnp.zeros((n,), jnp.float32))
    partsn = jnp.transpose(parts, (0, 2, 1))
    return _finalize(partsn, x, rootp, bias3)
